# baseline (speedup 1.0000x reference)
import jax
import jax.numpy as jnp
from jax import lax
from jax.experimental import pallas as pl
from jax.experimental.pallas import tpu as pltpu
from jax.experimental.pallas import tpu_sc as plsc

VOCAB = 1000000
DIM = 32
B = 4096
L = 200

NC = 2
NS = 16
NW = NC * NS

RCH = 16384
RGRID = -(-VOCAB // RCH)

NHALF = 4
BTOT = B * L
BH = BTOT // NHALF
B_PER_W = BH // NW
CHUNK = 160
NCHUNKS = B_PER_W // CHUNK
NBUF = 4
NSTEPS = NCHUNKS // NBUF

BCH = 4096


def _repack_tc_body(t_ref, x_ref):
    x_ref[:, pl.ds(0, DIM)] = t_ref[...].T


def _gather_body(x_hbm, idx_hbm, out_hbm, idx_v, rbuf, gsem):
    wid = lax.axis_index("s") * NC + lax.axis_index("c")
    base = wid * B_PER_W

    pltpu.sync_copy(idx_hbm.at[wid], idx_v)

    def start_gather(c, b):
        pltpu.make_async_copy(
            x_hbm.at[idx_v.at[c]], rbuf.at[b], gsem.at[b]
        ).start()

    def drain_chunk(c, b):
        pltpu.make_async_copy(
            x_hbm.at[idx_v.at[c]], rbuf.at[b], gsem.at[b]
        ).wait()
        pltpu.sync_copy(
            rbuf.at[b, :, pl.ds(0, DIM)],
            out_hbm.at[pl.ds(base + c * CHUNK, CHUNK), pl.ds(0, DIM)],
        )

    for b in range(NBUF):
        start_gather(b, b)

    @pl.loop(0, NSTEPS - 1)
    def _steady(i):
        for b in range(NBUF):
            c = i * NBUF + b
            drain_chunk(c, b)
            start_gather(c + NBUF, b)

    for b in range(NBUF):
        drain_chunk((NSTEPS - 1) * NBUF + b, b)


def _format_tc_body(r_ref, o_ref):
    y = r_ref[...].T
    o_ref[0] = y[0:DIM, 0:BCH]
    o_ref[1] = y[0:DIM, BCH:]


def _format_tc_body2(o_prev, r_ref, o_ref):
    del o_prev
    y = r_ref[...].T
    o_ref[0] = y[0:DIM, 0:BCH]
    o_ref[1] = y[0:DIM, BCH:]


@jax.jit
def _run(table, idx3d):
    repack = pl.pallas_call(
        _repack_tc_body,
        grid=(RGRID,),
        in_specs=[pl.BlockSpec((DIM, RCH), lambda i: (0, i))],
        out_specs=pl.BlockSpec((RCH, 128), lambda i: (i, 0)),
        out_shape=jax.ShapeDtypeStruct((VOCAB, 128), jnp.float32),
    )
    x = repack(table.T)

    mesh = plsc.VectorSubcoreMesh(core_axis_name="c", subcore_axis_name="s")

    def make_gather():
        return pl.kernel(
            _gather_body,
            out_type=jax.ShapeDtypeStruct((BH, 128), jnp.float32),
            mesh=mesh,
            scratch_types=[
                pltpu.VMEM((NCHUNKS, CHUNK), jnp.int32),
                pltpu.VMEM((NBUF, CHUNK, 128), jnp.float32),
                pltpu.SemaphoreType.DMA((NBUF,)),
            ],
            compiler_params=pltpu.CompilerParams(use_tc_tiling_on_sc=False),
        )

    rows = [make_gather()(x, idx3d[h]) for h in range(NHALF)]

    nblk = BH // (2 * BCH)
    fmt1 = pl.pallas_call(
        _format_tc_body,
        grid=(nblk,),
        in_specs=[pl.BlockSpec((2 * BCH, 128), lambda l: (l, 0))],
        out_specs=pl.BlockSpec((2, DIM, BCH), lambda l: (l, 0, 0)),
        out_shape=jax.ShapeDtypeStruct((L, DIM, B), jnp.float32),
    )
    out3 = fmt1(rows[0])
    for h in range(1, NHALF):
        fmt_h = pl.pallas_call(
            _format_tc_body2,
            grid=(nblk,),
            in_specs=[
                pl.BlockSpec(memory_space=pl.ANY),
                pl.BlockSpec((2 * BCH, 128), lambda l: (l, 0)),
            ],
            out_specs=pl.BlockSpec(
                (2, DIM, BCH), lambda l, h=h: (l + h * nblk, 0, 0)
            ),
            out_shape=jax.ShapeDtypeStruct((L, DIM, B), jnp.float32),
            input_output_aliases={0: 0},
        )
        out3 = fmt_h(out3, rows[h])
    return out3.transpose(2, 0, 1)


def kernel(inputs, table):
    idx3d = inputs.astype(jnp.int32).T.reshape(NHALF, NW, NCHUNKS, CHUNK)
    return _run(table, idx3d)


# speedup vs baseline: 1.0057x; 1.0057x over previous
import jax
import jax.numpy as jnp
from jax import lax
from jax.experimental import pallas as pl
from jax.experimental.pallas import tpu as pltpu
from jax.experimental.pallas import tpu_sc as plsc

VOCAB = 1000000
DIM = 32
B = 4096
L = 200

NC = 2
NS = 16
NW = NC * NS

RCH = 16384
RGRID = -(-VOCAB // RCH)

NHALF = 2
BTOT = B * L
BH = BTOT // NHALF
B_PER_W = BH // NW
CHUNK = 160
NCHUNKS = B_PER_W // CHUNK
NBUF = 4
NSTEPS = NCHUNKS // NBUF

BCH = 4096


def _repack_tc_body(t_ref, x_ref):
    x_ref[:, pl.ds(0, DIM)] = t_ref[...].T


def _gather_body(x_hbm, idx_hbm, out_hbm, idx_v, rbuf, gsem):
    wid = lax.axis_index("s") * NC + lax.axis_index("c")
    base = wid * B_PER_W

    pltpu.sync_copy(idx_hbm.at[wid], idx_v)

    def start_gather(c, b):
        pltpu.make_async_copy(
            x_hbm.at[idx_v.at[c]], rbuf.at[b], gsem.at[b]
        ).start()

    def drain_chunk(c, b):
        pltpu.make_async_copy(
            x_hbm.at[idx_v.at[c]], rbuf.at[b], gsem.at[b]
        ).wait()
        pltpu.sync_copy(
            rbuf.at[b, :, pl.ds(0, DIM)],
            out_hbm.at[pl.ds(base + c * CHUNK, CHUNK), pl.ds(0, DIM)],
        )

    for b in range(NBUF):
        start_gather(b, b)

    @pl.loop(0, NSTEPS - 1)
    def _steady(i):
        for b in range(NBUF):
            c = i * NBUF + b
            drain_chunk(c, b)
            start_gather(c + NBUF, b)

    for b in range(NBUF):
        drain_chunk((NSTEPS - 1) * NBUF + b, b)


def _format_tc_body(r_ref, o_ref):
    y = r_ref[...].T
    o_ref[0] = y[0:DIM, 0:BCH]
    o_ref[1] = y[0:DIM, BCH:]


def _format_tc_body2(o_prev, r_ref, o_ref):
    del o_prev
    y = r_ref[...].T
    o_ref[0] = y[0:DIM, 0:BCH]
    o_ref[1] = y[0:DIM, BCH:]


@jax.jit
def _run(table, idx3d):
    repack = pl.pallas_call(
        _repack_tc_body,
        grid=(RGRID,),
        in_specs=[pl.BlockSpec((DIM, RCH), lambda i: (0, i))],
        out_specs=pl.BlockSpec((RCH, 128), lambda i: (i, 0)),
        out_shape=jax.ShapeDtypeStruct((VOCAB, 128), jnp.float32),
    )
    x = repack(table.T)

    mesh = plsc.VectorSubcoreMesh(core_axis_name="c", subcore_axis_name="s")

    def make_gather():
        return pl.kernel(
            _gather_body,
            out_type=jax.ShapeDtypeStruct((BH, 128), jnp.float32),
            mesh=mesh,
            scratch_types=[
                pltpu.VMEM((NCHUNKS, CHUNK), jnp.int32),
                pltpu.VMEM((NBUF, CHUNK, 128), jnp.float32),
                pltpu.SemaphoreType.DMA((NBUF,)),
            ],
            compiler_params=pltpu.CompilerParams(use_tc_tiling_on_sc=False),
        )

    rows = [make_gather()(x, idx3d[h]) for h in range(NHALF)]

    nblk = BH // (2 * BCH)
    fmt1 = pl.pallas_call(
        _format_tc_body,
        grid=(nblk,),
        in_specs=[pl.BlockSpec((2 * BCH, 128), lambda l: (l, 0))],
        out_specs=pl.BlockSpec((2, DIM, BCH), lambda l: (l, 0, 0)),
        out_shape=jax.ShapeDtypeStruct((L, DIM, B), jnp.float32),
    )
    out3 = fmt1(rows[0])
    for h in range(1, NHALF):
        fmt_h = pl.pallas_call(
            _format_tc_body2,
            grid=(nblk,),
            in_specs=[
                pl.BlockSpec(memory_space=pl.ANY),
                pl.BlockSpec((2 * BCH, 128), lambda l: (l, 0)),
            ],
            out_specs=pl.BlockSpec(
                (2, DIM, BCH), lambda l, h=h: (l + h * nblk, 0, 0)
            ),
            out_shape=jax.ShapeDtypeStruct((L, DIM, B), jnp.float32),
            input_output_aliases={0: 0},
        )
        out3 = fmt_h(out3, rows[h])
    return out3.transpose(2, 0, 1)


def kernel(inputs, table):
    idx3d = inputs.astype(jnp.int32).T.reshape(NHALF, NW, NCHUNKS, CHUNK)
    return _run(table, idx3d)


# speedup vs baseline: 1.0141x; 1.0084x over previous
import jax
import jax.numpy as jnp
from jax import lax
from jax.experimental import pallas as pl
from jax.experimental.pallas import tpu as pltpu
from jax.experimental.pallas import tpu_sc as plsc

VOCAB = 1000000
DIM = 32
B = 4096
L = 200

NC = 2
NS = 16
NW = NC * NS

RCH = 32768
RGRID = -(-VOCAB // RCH)

NHALF = 2
BTOT = B * L
BH = BTOT // NHALF
B_PER_W = BH // NW
CHUNK = 160
NCHUNKS = B_PER_W // CHUNK
NBUF = 4
NSTEPS = NCHUNKS // NBUF

BCH = 4096


def _repack_tc_body(t_ref, x_ref):
    x_ref[:, pl.ds(0, DIM)] = t_ref[...].T


def _gather_body(x_hbm, idx_hbm, out_hbm, idx_v, rbuf, gsem):
    wid = lax.axis_index("s") * NC + lax.axis_index("c")
    base = wid * B_PER_W

    pltpu.sync_copy(idx_hbm.at[wid], idx_v)

    def start_gather(c, b):
        pltpu.make_async_copy(
            x_hbm.at[idx_v.at[c]], rbuf.at[b], gsem.at[b]
        ).start()

    def drain_chunk(c, b):
        pltpu.make_async_copy(
            x_hbm.at[idx_v.at[c]], rbuf.at[b], gsem.at[b]
        ).wait()
        pltpu.sync_copy(
            rbuf.at[b, :, pl.ds(0, DIM)],
            out_hbm.at[pl.ds(base + c * CHUNK, CHUNK), pl.ds(0, DIM)],
        )

    for b in range(NBUF):
        start_gather(b, b)

    @pl.loop(0, NSTEPS - 1)
    def _steady(i):
        for b in range(NBUF):
            c = i * NBUF + b
            drain_chunk(c, b)
            start_gather(c + NBUF, b)

    for b in range(NBUF):
        drain_chunk((NSTEPS - 1) * NBUF + b, b)


def _format_tc_body(r_ref, o_ref):
    y = r_ref[...].T
    o_ref[0] = y[0:DIM, 0:BCH]
    o_ref[1] = y[0:DIM, BCH:]


def _format_tc_body2(o_prev, r_ref, o_ref):
    del o_prev
    y = r_ref[...].T
    o_ref[0] = y[0:DIM, 0:BCH]
    o_ref[1] = y[0:DIM, BCH:]


@jax.jit
def _run(table, idx3d):
    repack = pl.pallas_call(
        _repack_tc_body,
        grid=(RGRID,),
        in_specs=[pl.BlockSpec((DIM, RCH), lambda i: (0, i))],
        out_specs=pl.BlockSpec((RCH, 128), lambda i: (i, 0)),
        out_shape=jax.ShapeDtypeStruct((VOCAB, 128), jnp.float32),
    )
    x = repack(table.T)

    mesh = plsc.VectorSubcoreMesh(core_axis_name="c", subcore_axis_name="s")

    def make_gather():
        return pl.kernel(
            _gather_body,
            out_type=jax.ShapeDtypeStruct((BH, 128), jnp.float32),
            mesh=mesh,
            scratch_types=[
                pltpu.VMEM((NCHUNKS, CHUNK), jnp.int32),
                pltpu.VMEM((NBUF, CHUNK, 128), jnp.float32),
                pltpu.SemaphoreType.DMA((NBUF,)),
            ],
            compiler_params=pltpu.CompilerParams(use_tc_tiling_on_sc=False),
        )

    rows = [make_gather()(x, idx3d[h]) for h in range(NHALF)]

    nblk = BH // (2 * BCH)
    fmt1 = pl.pallas_call(
        _format_tc_body,
        grid=(nblk,),
        in_specs=[pl.BlockSpec((2 * BCH, 128), lambda l: (l, 0))],
        out_specs=pl.BlockSpec((2, DIM, BCH), lambda l: (l, 0, 0)),
        out_shape=jax.ShapeDtypeStruct((L, DIM, B), jnp.float32),
    )
    out3 = fmt1(rows[0])
    for h in range(1, NHALF):
        fmt_h = pl.pallas_call(
            _format_tc_body2,
            grid=(nblk,),
            in_specs=[
                pl.BlockSpec(memory_space=pl.ANY),
                pl.BlockSpec((2 * BCH, 128), lambda l: (l, 0)),
            ],
            out_specs=pl.BlockSpec(
                (2, DIM, BCH), lambda l, h=h: (l + h * nblk, 0, 0)
            ),
            out_shape=jax.ShapeDtypeStruct((L, DIM, B), jnp.float32),
            input_output_aliases={0: 0},
        )
        out3 = fmt_h(out3, rows[h])
    return out3.transpose(2, 0, 1)


def kernel(inputs, table):
    idx3d = inputs.astype(jnp.int32).T.reshape(NHALF, NW, NCHUNKS, CHUNK)
    return _run(table, idx3d)
